# Initial kernel scaffold; baseline (speedup 1.0000x reference)
import jax
import jax.numpy as jnp
from jax.experimental import pallas as pl


def kernel(center_word, pos_words, neg_words, in_table, out_table):
    raise NotImplementedError("write your pallas kernel here")


# speedup vs baseline: 7.0967x; 7.0967x over previous
import functools

import jax
import jax.numpy as jnp
from jax import lax
from jax.experimental import pallas as pl
from jax.experimental.pallas import tpu as pltpu
from jax.experimental.pallas import tpu_sc as plsc

VOCAB = 100000
DIM = 200
B = 4096
N_POS = 20
N_NEG = 200
K = 224
KH = K // 2
NC = 2
NS = 16
NW = NC * NS
BPW = B // NW
LANES = 16
NCHUNK = DIM // LANES
TAIL_OFF = DIM - LANES
PAD_SCORE = 30.0


_mesh = plsc.VectorSubcoreMesh(core_axis_name="c", subcore_axis_name="s")


@functools.partial(
    pl.kernel,
    mesh=_mesh,
    out_type=jax.ShapeDtypeStruct((B, K), jnp.float32),
    compiler_params=pltpu.CompilerParams(
        needs_layout_passes=False, use_tc_tiling_on_sc=False),
    scratch_types=[
        pltpu.VMEM((BPW,), jnp.int32),
        pltpu.VMEM((BPW, DIM), jnp.float32),
        pltpu.VMEM((2, KH), jnp.int32),
        pltpu.VMEM((K, DIM), jnp.float32),
        pltpu.VMEM((K,), jnp.float32),
        pltpu.SemaphoreType.DMA,
    ],
)
def _sc_scores(idx_hbm, cw_hbm, in_t_hbm, out_t_hbm, out_hbm,
               cidx_v, crows_v, kidx_v, rows_v, sc_v, sem):
    wid = lax.axis_index("s") * NC + lax.axis_index("c")
    base = wid * BPW

    pltpu.sync_copy(cw_hbm.at[pl.ds(base, BPW)], cidx_v)
    pltpu.async_copy(in_t_hbm.at[cidx_v], crows_v, sem).wait()

    lane = lax.iota(jnp.int32, 16)
    tail_mask = lane >= (LANES - (DIM - NCHUNK * LANES))
    lane0 = lane == 0
    pad_mask = lane < (K - N_POS - N_NEG)
    pad_idx = lane + (N_POS + N_NEG)
    pad_val = jnp.full((16,), PAD_SCORE, jnp.float32)

    def body(b, _):
        pltpu.sync_copy(idx_hbm.at[base + b], kidx_v)
        c0 = pltpu.async_copy(out_t_hbm.at[kidx_v.at[0]],
                              rows_v.at[pl.ds(0, KH)], sem)
        c1 = pltpu.async_copy(out_t_hbm.at[kidx_v.at[1]],
                              rows_v.at[pl.ds(KH, KH)], sem)
        c0.wait()
        c1.wait()

        cvec = [crows_v[b, pl.ds(t * LANES, LANES)] for t in range(NCHUNK)]
        ctail = jnp.where(tail_mask, crows_v[b, pl.ds(TAIL_OFF, LANES)], 0.0)

        def row(j, _):
            acc = rows_v[j, pl.ds(0, LANES)] * cvec[0]
            for t in range(1, NCHUNK):
                acc += rows_v[j, pl.ds(t * LANES, LANES)] * cvec[t]
            acc += rows_v[j, pl.ds(TAIL_OFF, LANES)] * ctail
            s = jnp.sum(acc)
            s = jnp.where(j < N_POS, s, -s)
            plsc.store_scatter(sc_v, [jnp.full((16,), j, jnp.int32)],
                               jnp.full((16,), s), mask=lane0)
            return 0

        lax.fori_loop(0, N_POS + N_NEG, row, 0, unroll=4)
        plsc.store_scatter(sc_v, [pad_idx], pad_val, mask=pad_mask)
        pltpu.sync_copy(sc_v, out_hbm.at[base + b])
        return 0

    lax.fori_loop(0, BPW, body, 0)


def _loss_body(scores_ref, out_ref):
    i = pl.program_id(0)

    @pl.when(i == 0)
    def _init():
        out_ref[...] = jnp.zeros((1, 1), jnp.float32)

    ls = jax.nn.log_sigmoid(scores_ref[...])
    out_ref[...] += jnp.sum(ls).reshape(1, 1)

    @pl.when(i == pl.num_programs(0) - 1)
    def _fini():
        out_ref[...] = -out_ref[...] / B


def kernel(center_word, pos_words, neg_words, in_table, out_table):
    idx_all = jnp.concatenate(
        [pos_words, neg_words,
         jnp.zeros((B, K - N_POS - N_NEG), jnp.int32)], axis=1)
    idx_all = idx_all.reshape(B, 2, KH)

    scores = _sc_scores(idx_all, center_word, in_table, out_table)

    rows_blk = 256
    loss = pl.pallas_call(
        _loss_body,
        grid=(B // rows_blk,),
        in_specs=[pl.BlockSpec((rows_blk, K), lambda i: (i, 0))],
        out_specs=pl.BlockSpec((1, 1), lambda i: (0, 0)),
        out_shape=jax.ShapeDtypeStruct((1, 1), jnp.float32),
    )(scores)
    return loss[0, 0]


# speedup vs baseline: 8.9513x; 1.2613x over previous
import functools

import jax
import jax.numpy as jnp
from jax import lax
from jax.experimental import pallas as pl
from jax.experimental.pallas import tpu as pltpu
from jax.experimental.pallas import tpu_sc as plsc

VOCAB = 100000
DIM = 200
DPAD = 256
B = 4096
N_POS = 20
N_NEG = 200
K = 224
KH = K // 2
NGATHER = KH // 16
NC = 2
NS = 16
NW = NC * NS
BPW = B // NW
BBLK = 64
LANES = 16
NCHUNK = DIM // LANES
TAIL_OFF = DIM - LANES


_mesh = plsc.VectorSubcoreMesh(core_axis_name="c", subcore_axis_name="s")


@functools.partial(
    pl.kernel,
    mesh=_mesh,
    out_type=jax.ShapeDtypeStruct((B * K,), jnp.float32),
    compiler_params=pltpu.CompilerParams(
        needs_layout_passes=False, use_tc_tiling_on_sc=True),
    scratch_types=[
        pltpu.VMEM((BPW,), jnp.int32),
        pltpu.VMEM((BPW, DPAD), jnp.float32),
        pltpu.VMEM((BBLK, K), jnp.int32),
        pltpu.VMEM((2, KH, DPAD), jnp.float32),
        pltpu.VMEM((BBLK * K,), jnp.float32),
        pltpu.SemaphoreType.DMA((2,)),
        pltpu.SemaphoreType.DMA,
    ],
)
def _sc_scores(idx_hbm, cw_hbm, in_t_hbm, out_t_hbm, out_hbm,
               cidx_v, crows_v, kidx_v, rows_v, sc_v, gsem, ssem):
    wid = lax.axis_index("s") * NC + lax.axis_index("c")
    lane = lax.iota(jnp.int32, 16)
    tail_mask = lane >= (LANES - (DIM - NCHUNK * LANES))
    lane0 = lane == 0

    pltpu.sync_copy(cw_hbm.at[pl.ds(wid * BPW, BPW)], cidx_v)
    pltpu.async_copy(in_t_hbm.at[cidx_v], crows_v, ssem).wait()

    def fire(t):
        b1 = t >> 1
        h1 = t & 1
        for g in range(NGATHER):
            iv = kidx_v[b1, pl.ds(h1 * KH + g * 16, 16)]
            pltpu.async_copy(out_t_hbm.at[iv],
                             rows_v.at[h1, pl.ds(g * 16, 16)], gsem.at[h1])

    def drain(h):
        for g in range(NGATHER):
            pltpu.make_async_copy(out_t_hbm.at[lane],
                                  rows_v.at[h, pl.ds(g * 16, 16)],
                                  gsem.at[h]).wait()

    for c in range(BPW // BBLK):
        base = wid * BPW + c * BBLK

        pltpu.sync_copy(idx_hbm.at[pl.ds(base, BBLK)], kidx_v)
        fire(0)

        nt = 2 * BBLK

        def chunk(t, _):
            b = t >> 1
            h = t & 1

            @pl.when(t < nt - 1)
            def _prefetch():
                fire(t + 1)

            drain(h)

            bc = c * BBLK + b
            cvec = [crows_v[bc, pl.ds(u * LANES, LANES)]
                    for u in range(NCHUNK)]
            ctail = jnp.where(tail_mask,
                              crows_v[bc, pl.ds(TAIL_OFF, LANES)], 0.0)
            obase = jnp.full((16,), b * K + h * KH, jnp.int32)

            def row(j, _):
                acc = rows_v[h, j, pl.ds(0, LANES)] * cvec[0]
                for u in range(1, NCHUNK):
                    acc += rows_v[h, j, pl.ds(u * LANES, LANES)] * cvec[u]
                acc += rows_v[h, j, pl.ds(TAIL_OFF, LANES)] * ctail
                s = jnp.sum(acc)
                plsc.store_scatter(sc_v, [obase + j],
                                   jnp.full((16,), s), mask=lane0)
                return 0

            lax.fori_loop(0, KH, row, 0, unroll=4)
            return 0

        lax.fori_loop(0, nt, chunk, 0)
        pltpu.sync_copy(sc_v, out_hbm.at[pl.ds(base * K, BBLK * K)])


def _loss_body(scores_ref, out_ref):
    i = pl.program_id(0)

    @pl.when(i == 0)
    def _init():
        out_ref[...] = jnp.zeros((1, 1), jnp.float32)

    x = scores_ref[...]
    col = lax.broadcasted_iota(jnp.int32, x.shape, 1)
    x = jnp.where(col < N_POS, x, -x)
    ls = jnp.where(col < N_POS + N_NEG, jax.nn.log_sigmoid(x), 0.0)
    out_ref[...] += jnp.sum(ls).reshape(1, 1)

    @pl.when(i == pl.num_programs(0) - 1)
    def _fini():
        out_ref[...] = -out_ref[...] / B


def kernel(center_word, pos_words, neg_words, in_table, out_table):
    idx_all = jnp.concatenate(
        [pos_words, neg_words,
         jnp.zeros((B, K - N_POS - N_NEG), jnp.int32)], axis=1)
    in_pad = jnp.pad(in_table, ((0, 0), (0, DPAD - DIM)))
    out_pad = jnp.pad(out_table, ((0, 0), (0, DPAD - DIM)))

    scores = _sc_scores(idx_all, center_word, in_pad, out_pad)
    scores = scores.reshape(B, K)

    rows_blk = 256
    loss = pl.pallas_call(
        _loss_body,
        grid=(B // rows_blk,),
        in_specs=[pl.BlockSpec((rows_blk, K), lambda i: (i, 0))],
        out_specs=pl.BlockSpec((1, 1), lambda i: (0, 0)),
        out_shape=jax.ShapeDtypeStruct((1, 1), jnp.float32),
    )(scores)
    return loss[0, 0]
